# Initial kernel scaffold; baseline (speedup 1.0000x reference)
import jax
import jax.numpy as jnp
from jax.experimental import pallas as pl


def kernel(events):
    raise NotImplementedError("write your pallas kernel here")


# speedup vs baseline: 1.9992x; 1.9992x over previous
import functools

import jax
import jax.numpy as jnp
from jax import lax
from jax.experimental import pallas as pl
from jax.experimental.pallas import tpu as pltpu
from jax.experimental.pallas import tpu_sc as plsc

N_EVENTS = 4194304
NUM_SLICES = 10
DOWN_H = 180
DOWN_W = 320
GRID = NUM_SLICES * 2 * DOWN_H * DOWN_W

NC = 2
NS = 16
PT = N_EVENTS // (NC * NS)
C = 2048
ROWS = C // 128
SLICE_PER_TILE = GRID // NS
ZCHUNK = 7200


def _minmax_body(x_ref, mn_ref, mx_ref):
    i = pl.program_id(0)
    x = x_ref[...]
    mask = (lax.broadcasted_iota(jnp.int32, x.shape, 1) % 4) == 2
    pmin = jnp.min(jnp.where(mask, x, jnp.inf))
    pmax = jnp.max(jnp.where(mask, x, -jnp.inf))

    @pl.when(i == 0)
    def _():
        mn_ref[0, 0] = pmin
        mx_ref[0, 0] = pmax

    @pl.when(i != 0)
    def _():
        mn_ref[0, 0] = jnp.minimum(mn_ref[0, 0], pmin)
        mx_ref[0, 0] = jnp.maximum(mx_ref[0, 0], pmax)


_minmax = pl.pallas_call(
    _minmax_body,
    grid=(8,),
    in_specs=[pl.BlockSpec((2048, 1024), lambda i: (i, 0))],
    out_specs=[
        pl.BlockSpec(memory_space=pltpu.SMEM),
        pl.BlockSpec(memory_space=pltpu.SMEM),
    ],
    out_shape=[
        jax.ShapeDtypeStruct((1, 1), jnp.float32),
        jax.ShapeDtypeStruct((1, 1), jnp.float32),
    ],
)


_sc_mesh = plsc.VectorSubcoreMesh(core_axis_name="c", subcore_axis_name="s")


@functools.partial(
    pl.kernel,
    mesh=_sc_mesh,
    out_type=jax.ShapeDtypeStruct((NC * GRID,), jnp.float32),
    scratch_types=[
        pltpu.VMEM_SHARED((GRID,), jnp.float32),
        pltpu.VMEM((C * 4,), jnp.float32),
        pltpu.VMEM((ROWS, 128), jnp.int32),
        pltpu.VMEM((128,), jnp.float32),
        pltpu.VMEM((ZCHUNK,), jnp.float32),
        pltpu.VMEM((16,), jnp.float32),
        pltpu.SemaphoreType.DMA,
    ],
    compiler_params=pltpu.CompilerParams(needs_layout_passes=False),
)
def _sc_scatter(ev_hbm, mm_hbm, out_hbm, grid_sp, evbuf, idxbuf, onesbuf,
                zbuf, mmv, sem):
    cid = lax.axis_index("c")
    sid = lax.axis_index("s")

    pltpu.sync_copy(mm_hbm, mmv)
    mmvec = mmv[...]
    tmin = mmvec[0]
    tmax = mmvec[1]
    pred = tmax > tmin
    denom = jnp.where(pred, tmax - tmin, jnp.float32(1.0))

    zeros16 = jnp.zeros((16,), jnp.float32)
    ones16 = jnp.ones((16,), jnp.float32)

    def _fill_z(i, carry):
        zbuf[pl.ds(i * 16, 16)] = zeros16
        return carry

    lax.fori_loop(0, ZCHUNK // 16, _fill_z, 0)
    for r in range(8):
        onesbuf[pl.ds(r * 16, 16)] = ones16

    def _zero_grid(k, carry):
        pltpu.sync_copy(
            zbuf, grid_sp.at[pl.ds(sid * SLICE_PER_TILE + k * ZCHUNK, ZCHUNK)])
        return carry

    lax.fori_loop(0, SLICE_PER_TILE // ZCHUNK, _zero_grid, 0)
    plsc.subcore_barrier()

    lane4 = lax.iota(jnp.int32, 16) * 4
    tile_base = cid * (N_EVENTS // NC) + sid * PT

    def _chunk(ci, carry):
        base_f = (tile_base + ci * C) * 4
        pltpu.sync_copy(ev_hbm.at[pl.ds(base_f, C * 4)], evbuf)

        def _group(g, gcarry):
            fbase = lane4 + g * 64
            xv = plsc.load_gather(evbuf, [fbase])
            yv = plsc.load_gather(evbuf, [fbase + 1])
            tv = plsc.load_gather(evbuf, [fbase + 2])
            pv = plsc.load_gather(evbuf, [fbase + 3])
            tn = jnp.where(pred, (tv - tmin) / denom * 50.0, tv)
            s = jnp.clip((tn / 5.0).astype(jnp.int32), 0, NUM_SLICES - 1)
            xq = jnp.clip((xv / 4.0).astype(jnp.int32), 0, DOWN_W - 1)
            yq = jnp.clip((yv / 4.0).astype(jnp.int32), 0, DOWN_H - 1)
            pn = jnp.where(pv <= 0.0, 1, 0)
            flat = (s * 2 + pn) * (DOWN_H * DOWN_W) + yq * DOWN_W + xq
            r = g // 8
            col = (g % 8) * 16
            idxbuf[r, pl.ds(col, 16)] = flat
            return gcarry

        lax.fori_loop(0, C // 16, _group, 0)

        copies = [
            pltpu.async_copy(onesbuf, grid_sp.at[idxbuf.at[r]], sem)
            for r in range(ROWS)
        ]
        for cp in copies:
            cp.wait()
        return carry

    lax.fori_loop(0, PT // C, _chunk, 0)
    plsc.subcore_barrier()

    def _flush(k, carry):
        off = sid * SLICE_PER_TILE + k * ZCHUNK
        pltpu.sync_copy(grid_sp.at[pl.ds(off, ZCHUNK)], zbuf)
        pltpu.sync_copy(zbuf, out_hbm.at[pl.ds(cid * GRID + off, ZCHUNK)])
        return carry

    lax.fori_loop(0, SLICE_PER_TILE // ZCHUNK, _flush, 0)


def _merge_body(a_ref, o_ref):
    o_ref[...] = jnp.maximum(a_ref[0], a_ref[1])


_merge = pl.pallas_call(
    _merge_body,
    out_shape=jax.ShapeDtypeStruct((9000, 128), jnp.float32),
)


@jax.jit
def kernel(events):
    ev_flat = events.reshape(-1)
    tmin, tmax = _minmax(ev_flat.reshape(16384, 1024))
    mm = jnp.zeros((16,), jnp.float32).at[0].set(tmin[0, 0]).at[1].set(tmax[0, 0])
    halves = _sc_scatter(ev_flat, mm)
    merged = _merge(halves.reshape(2, 9000, 128))
    return merged.reshape(NUM_SLICES * 2, DOWN_H, DOWN_W)
